# baseline (speedup 1.0000x reference)
import functools

import jax
import jax.numpy as jnp
from jax import lax
from jax.experimental import pallas as pl
from jax.experimental.pallas import tpu as pltpu
from jax.experimental.pallas import tpu_sc as plsc

_B, _L = 4096, 200
_V, _Din, _Dout = 500, 100, 30
_N = _B * _L

_NC, _NS = 2, 16
_NW = _NC * _NS
_PER_W = _N // _NW
_CH = 128
_NCH = _PER_W // _CH
_SUP = 2
_BIG = _SUP * _CH
_NSUP = _PER_W // _BIG

_VP, _DinP, _TW = 512, 128, 128


def _table_body(emb_ref, wt_ref, b_ref, out_ref):
    acc = lax.dot_general(
        emb_ref[...], wt_ref[...],
        (((1,), (0,)), ((), ())),
        preferred_element_type=jnp.float32,
    )
    out_ref[...] = jnp.tanh(acc + b_ref[...])


def _make_table(emb, W, b):
    emb_p = jnp.zeros((_VP, _DinP), jnp.float32).at[:_V, :_Din].set(emb)
    wt_p = jnp.zeros((_DinP, _TW), jnp.float32).at[:_Din, :_Dout].set(W.T)
    b_p = jnp.zeros((1, _TW), jnp.float32).at[0, :_Dout].set(b)
    return pl.pallas_call(
        _table_body,
        out_shape=jax.ShapeDtypeStruct((_VP, _TW), jnp.float32),
    )(emb_p, wt_p, b_p)


@functools.partial(
    pl.kernel,
    out_type=jax.ShapeDtypeStruct((_N, _TW), jnp.float32),
    mesh=plsc.VectorSubcoreMesh(core_axis_name="c", subcore_axis_name="s"),
    compiler_params=pltpu.CompilerParams(use_tc_tiling_on_sc=True),
    scratch_types=[
        pltpu.VMEM((_NCH, _CH), jnp.int32),
        pltpu.VMEM((_BIG, _TW), jnp.float32),
        pltpu.VMEM((_BIG, _TW), jnp.float32),
        pltpu.VMEM_SHARED((_VP, _TW), jnp.float32),
        pltpu.SemaphoreType.DMA,
        pltpu.SemaphoreType.DMA,
    ],
)
def _gather(table_hbm, idx_hbm, out_hbm, idx_v, rows_a, rows_b, tab_s,
            sem_a, sem_b):
    sid = lax.axis_index("s")
    wid = sid * _NC + lax.axis_index("c")
    base = wid * _PER_W

    @pl.when(sid == 0)
    def _():
        pltpu.sync_copy(table_hbm, tab_s)
    plsc.subcore_barrier()
    pltpu.sync_copy(idx_hbm.at[wid], idx_v)

    def fire(s, rows, sem):
        for j in range(_SUP):
            pltpu.async_copy(
                tab_s.at[idx_v.at[s * _SUP + j]],
                rows.at[pl.ds(j * _CH, _CH)], sem)

    def drain(rows, sem):
        pltpu.make_async_copy(out_hbm.at[pl.ds(0, _BIG)], rows, sem).wait()

    def put(s, rows):
        pltpu.sync_copy(rows, out_hbm.at[pl.ds(base + s * _BIG, _BIG)])

    fire(0, rows_a, sem_a)

    @pl.loop(0, _NSUP, step=2)
    def _body(s):
        fire(s + 1, rows_b, sem_b)
        drain(rows_a, sem_a)
        put(s, rows_a)

        @pl.when(s + 2 < _NSUP)
        def _():
            fire(s + 2, rows_a, sem_a)

        drain(rows_b, sem_b)
        put(s + 1, rows_b)


def kernel(x, emb, W, b):
    table = _make_table(emb, W, b)
    idx = x.astype(jnp.int32).reshape(_NW, _NCH, _CH)
    out = _gather(table, idx)
    return out[:, :_Dout].reshape(_B, _L, _Dout)


# speedup vs baseline: 1.4947x; 1.4947x over previous
import functools

import jax
import jax.numpy as jnp
from jax import lax
from jax.experimental import pallas as pl
from jax.experimental.pallas import tpu as pltpu
from jax.experimental.pallas import tpu_sc as plsc

_B, _L = 4096, 200
_V, _Din, _Dout = 500, 100, 30
_N = _B * _L

_NC, _NS = 2, 16
_NW = _NC * _NS
_PER_W = _N // _NW
_CH = 128
_NCH = _PER_W // _CH
_SUP = 10
_BIG = _SUP * _CH
_NSUP = _PER_W // _BIG

_VP, _DinP, _DoutP, _TW = 512, 128, 32, 128


def _table_body(emb_ref, wt_ref, b_ref, out_ref):
    acc = lax.dot_general(
        emb_ref[...], wt_ref[...],
        (((1,), (0,)), ((), ())),
        preferred_element_type=jnp.float32,
    )
    out_ref[...] = jnp.tanh(acc + b_ref[...])


def _make_table(emb, W, b):
    emb_p = jnp.zeros((_VP, _DinP), jnp.float32).at[:_V, :_Din].set(emb)
    wt_p = jnp.zeros((_DinP, _DoutP), jnp.float32).at[:_Din, :_Dout].set(W.T)
    b_p = jnp.zeros((1, _DoutP), jnp.float32).at[0, :_Dout].set(b)
    return pl.pallas_call(
        _table_body,
        out_shape=jax.ShapeDtypeStruct((_VP, _DoutP), jnp.float32),
    )(emb_p, wt_p, b_p)


@functools.partial(
    pl.kernel,
    out_type=jax.ShapeDtypeStruct((_N, _TW), jnp.float32),
    mesh=plsc.VectorSubcoreMesh(core_axis_name="c", subcore_axis_name="s"),
    compiler_params=pltpu.CompilerParams(use_tc_tiling_on_sc=False),
    scratch_types=[
        pltpu.VMEM((_NCH, _CH), jnp.int32),
        pltpu.VMEM((_BIG, _DoutP), jnp.float32),
        pltpu.VMEM((_BIG, _DoutP), jnp.float32),
        pltpu.VMEM_SHARED((_VP, _DoutP), jnp.float32),
        pltpu.SemaphoreType.DMA,
        pltpu.SemaphoreType.DMA,
    ],
)
def _gather(table_hbm, idx_hbm, out_hbm, idx_v, rows_a, rows_b, tab_s,
            sem_a, sem_b):
    sid = lax.axis_index("s")
    wid = sid * _NC + lax.axis_index("c")
    base = wid * _PER_W

    @pl.when(sid == 0)
    def _():
        pltpu.sync_copy(table_hbm, tab_s)
    plsc.subcore_barrier()
    pltpu.sync_copy(idx_hbm.at[wid], idx_v)

    def fire(s, rows, sem):
        for j in range(_SUP):
            pltpu.async_copy(
                tab_s.at[idx_v.at[s * _SUP + j]],
                rows.at[pl.ds(j * _CH, _CH)], sem)

    def drain(rows, sem):
        pltpu.make_async_copy(out_hbm.at[pl.ds(0, _BIG), pl.ds(0, _DoutP)],
                              rows, sem).wait()

    def put(s, rows):
        pltpu.sync_copy(rows, out_hbm.at[pl.ds(base + s * _BIG, _BIG),
                                         pl.ds(0, _DoutP)])

    fire(0, rows_a, sem_a)

    @pl.loop(0, _NSUP, step=2)
    def _body(s):
        fire(s + 1, rows_b, sem_b)
        drain(rows_a, sem_a)
        put(s, rows_a)

        @pl.when(s + 2 < _NSUP)
        def _():
            fire(s + 2, rows_a, sem_a)

        drain(rows_b, sem_b)
        put(s + 1, rows_b)


def kernel(x, emb, W, b):
    table = _make_table(emb, W, b)
    idx = x.astype(jnp.int32).reshape(_NW, _NCH, _CH)
    out = _gather(table, idx)
    return out[:, :_Dout].reshape(_B, _L, _Dout)
